# baseline (speedup 1.0000x reference)
import functools

import jax
import jax.numpy as jnp
from jax import lax
from jax.experimental import pallas as pl
from jax.experimental.pallas import tpu as pltpu
from jax.experimental.pallas import tpu_sc as plsc

VOCAB_N = 100000
CTX_N = 20
EMB_N = 64
HID_N = 512
BATCH_N = 1024

_NC = 2
_NS = 16
_NW = _NC * _NS
_TOT = BATCH_N * CTX_N
_CHUNK = 128
_NROWS = _TOT // _CHUNK
_CPW = _NROWS // _NW

@functools.cache
def _sc_gather_fn():
    mesh = plsc.VectorSubcoreMesh(core_axis_name="c", subcore_axis_name="s")

    @functools.partial(
        pl.kernel,
        mesh=mesh,
        out_type=jax.ShapeDtypeStruct((_NW, _CPW, _CHUNK, EMB_N), jnp.float32),
        scratch_types=[
            pltpu.VMEM((_CPW, _CHUNK), jnp.int32),
            pltpu.VMEM((_CPW, _CHUNK, EMB_N), jnp.float32),
            pltpu.SemaphoreType.DMA,
        ],
        compiler_params=pltpu.CompilerParams(use_tc_tiling_on_sc=False),
    )
    def _sc_gather(table_hbm, idx_hbm, out_hbm, idx_v, rows_v, sem):
        wid = lax.axis_index("s") * _NC + lax.axis_index("c")
        pltpu.sync_copy(idx_hbm.at[wid], idx_v)
        copies = [
            pltpu.async_copy(table_hbm.at[idx_v.at[i]], rows_v.at[i], sem)
            for i in range(_CPW)
        ]
        for c in copies:
            c.wait()
        pltpu.sync_copy(rows_v, out_hbm.at[wid])

    return _sc_gather


_BN = 1024
_NB = (VOCAB_N + _BN - 1) // _BN


def _h_body(flat_ref, w1_ref, b1_ref, h_ref):
    h = jnp.dot(flat_ref[...].astype(jnp.bfloat16),
                w1_ref[...].astype(jnp.bfloat16),
                preferred_element_type=jnp.float32)
    h_ref[...] = jnp.maximum(h + b1_ref[...], 0.0).astype(jnp.bfloat16)


_h_layer = pl.pallas_call(
    _h_body,
    out_shape=jax.ShapeDtypeStruct((BATCH_N, HID_N), jnp.bfloat16),
)


def _out_body(h_ref, w2_ref, b2_ref, out_ref):
    out_ref[...] = (
        jnp.dot(h_ref[...], w2_ref[...].astype(jnp.bfloat16),
                preferred_element_type=jnp.float32)
        + b2_ref[...]
    )


_out_layer = pl.pallas_call(
    _out_body,
    grid=(_NB,),
    in_specs=[
        pl.BlockSpec((BATCH_N, HID_N), lambda j: (0, 0)),
        pl.BlockSpec((HID_N, _BN), lambda j: (0, j)),
        pl.BlockSpec((1, _BN), lambda j: (0, j)),
    ],
    out_specs=pl.BlockSpec((BATCH_N, _BN), lambda j: (0, j)),
    out_shape=jax.ShapeDtypeStruct((BATCH_N, VOCAB_N), jnp.float32),
)


def kernel(x, emb_table, W1, b1, W2, b2):
    idx = x.astype(jnp.int32).reshape(_NW, _CPW, _CHUNK)
    flat4 = _sc_gather_fn()(emb_table, idx)
    flat = flat4.reshape(BATCH_N, CTX_N * EMB_N)
    h = _h_layer(flat, W1, b1.reshape(1, HID_N))
    return _out_layer(h, W2, b2.reshape(1, VOCAB_N))


# speedup vs baseline: 1.0388x; 1.0388x over previous
import functools

import jax
import jax.numpy as jnp
from jax import lax
from jax.experimental import pallas as pl
from jax.experimental.pallas import tpu as pltpu
from jax.experimental.pallas import tpu_sc as plsc

VOCAB_N = 100000
CTX_N = 20
EMB_N = 64
HID_N = 512
BATCH_N = 1024

_NC = 2
_NS = 16
_NW = _NC * _NS
_TOT = BATCH_N * CTX_N
_CHUNK = 128
_NROWS = _TOT // _CHUNK
_CPW = _NROWS // _NW

@functools.cache
def _sc_gather_fn():
    mesh = plsc.VectorSubcoreMesh(core_axis_name="c", subcore_axis_name="s")

    @functools.partial(
        pl.kernel,
        mesh=mesh,
        out_type=jax.ShapeDtypeStruct((_NW, _CPW, _CHUNK, EMB_N), jnp.float32),
        scratch_types=[
            pltpu.VMEM((_CPW, _CHUNK), jnp.int32),
            pltpu.VMEM((_CPW, _CHUNK, EMB_N), jnp.float32),
            pltpu.SemaphoreType.DMA,
        ],
        compiler_params=pltpu.CompilerParams(use_tc_tiling_on_sc=False),
    )
    def _sc_gather(table_hbm, idx_hbm, out_hbm, idx_v, rows_v, sem):
        wid = lax.axis_index("s") * _NC + lax.axis_index("c")
        pltpu.sync_copy(idx_hbm.at[wid], idx_v)
        copies = [
            pltpu.async_copy(table_hbm.at[idx_v.at[i]], rows_v.at[i], sem)
            for i in range(_CPW)
        ]
        for c in copies:
            c.wait()
        pltpu.sync_copy(rows_v, out_hbm.at[wid])

    return _sc_gather


_BN = 4096
_NB = (VOCAB_N + _BN - 1) // _BN


def _h_body(flat_ref, w1_ref, b1_ref, h_ref):
    h = jnp.dot(flat_ref[...].astype(jnp.bfloat16),
                w1_ref[...].astype(jnp.bfloat16),
                preferred_element_type=jnp.float32)
    h_ref[...] = jnp.maximum(h + b1_ref[...], 0.0).astype(jnp.bfloat16)


_h_layer = pl.pallas_call(
    _h_body,
    out_shape=jax.ShapeDtypeStruct((BATCH_N, HID_N), jnp.bfloat16),
)


def _out_body(h_ref, w2_ref, b2_ref, out_ref):
    out_ref[...] = (
        jnp.dot(h_ref[...], w2_ref[...].astype(jnp.bfloat16),
                preferred_element_type=jnp.float32)
        + b2_ref[...]
    )


_out_layer = pl.pallas_call(
    _out_body,
    grid=(_NB,),
    in_specs=[
        pl.BlockSpec((BATCH_N, HID_N), lambda j: (0, 0)),
        pl.BlockSpec((HID_N, _BN), lambda j: (0, j)),
        pl.BlockSpec((1, _BN), lambda j: (0, j)),
    ],
    out_specs=pl.BlockSpec((BATCH_N, _BN), lambda j: (0, j)),
    out_shape=jax.ShapeDtypeStruct((BATCH_N, VOCAB_N), jnp.float32),
)


def kernel(x, emb_table, W1, b1, W2, b2):
    idx = x.astype(jnp.int32).reshape(_NW, _CPW, _CHUNK)
    flat4 = _sc_gather_fn()(emb_table, idx)
    flat = flat4.reshape(BATCH_N, CTX_N * EMB_N)
    h = _h_layer(flat, W1, b1.reshape(1, HID_N))
    return _out_layer(h, W2, b2.reshape(1, VOCAB_N))
